# baseline (speedup 1.0000x reference)
import functools

import jax
import jax.numpy as jnp
from jax import lax
from jax.experimental import pallas as pl
from jax.experimental.pallas import tpu as pltpu
from jax.experimental.pallas import tpu_sc as plsc

B = 16384
NC, NS, L = 2, 16, 16
NW = NC * NS
BPW = B // NW
CHUNK = 128
NCHUNK = BPW // CHUNK


@functools.partial(
    pl.kernel,
    mesh=plsc.VectorSubcoreMesh(core_axis_name="c", subcore_axis_name="s"),
    out_type=jax.ShapeDtypeStruct((B,), jnp.float32),
    scratch_types=[
        pltpu.VMEM((BPW,), jnp.int32),
        pltpu.VMEM((BPW,), jnp.float32),
        pltpu.SemaphoreType.DMA,
    ],
)
def _gather_sigmoid(idx_hbm, alpha_hbm, out_hbm, idx_v, vals_v, sem):
    wid = lax.axis_index("s") * NC + lax.axis_index("c")
    base = wid * BPW

    pltpu.sync_copy(idx_hbm.at[pl.ds(base, BPW)], idx_v)

    pltpu.async_copy(alpha_hbm.at[idx_v], vals_v, sem).wait()

    one = jnp.full((L,), 1.0, dtype=jnp.float32)

    def body(i, _):
        x = vals_v[pl.ds(i * L, L)]
        vals_v[pl.ds(i * L, L)] = one / (one + jnp.exp(-x))
        return 0

    lax.fori_loop(0, BPW // L, body, 0)

    pltpu.sync_copy(vals_v, out_hbm.at[pl.ds(base, BPW)])


def kernel(idx, alpha):
    return _gather_sigmoid(idx.astype(jnp.int32), alpha)


# speedup vs baseline: 1.0010x; 1.0010x over previous
import functools

import jax
import jax.numpy as jnp
from jax import lax
from jax.experimental import pallas as pl
from jax.experimental.pallas import tpu as pltpu
from jax.experimental.pallas import tpu_sc as plsc

B = 16384
NC, NS, L = 1, 16, 16
NW = NC * NS
BPW = B // NW


@functools.partial(
    pl.kernel,
    mesh=plsc.VectorSubcoreMesh(core_axis_name="c", subcore_axis_name="s",
                                num_cores=NC),
    out_type=jax.ShapeDtypeStruct((B,), jnp.float32),
    scratch_types=[
        pltpu.VMEM((BPW,), jnp.int32),
        pltpu.VMEM((BPW,), jnp.float32),
        pltpu.SemaphoreType.DMA,
    ],
)
def _gather_sigmoid(idx_hbm, alpha_hbm, out_hbm, idx_v, vals_v, sem):
    wid = lax.axis_index("s") * NC + lax.axis_index("c")
    base = wid * BPW

    pltpu.sync_copy(idx_hbm.at[pl.ds(base, BPW)], idx_v)

    pltpu.async_copy(alpha_hbm.at[idx_v], vals_v, sem).wait()

    one = jnp.full((L,), 1.0, dtype=jnp.float32)
    for i in range(BPW // L):
        x = vals_v[pl.ds(i * L, L)]
        vals_v[pl.ds(i * L, L)] = one / (one + jnp.exp(-x))

    pltpu.sync_copy(vals_v, out_hbm.at[pl.ds(base, BPW)])


def kernel(idx, alpha):
    return _gather_sigmoid(idx.astype(jnp.int32), alpha)


# speedup vs baseline: 1.0194x; 1.0184x over previous
import functools

import jax
import jax.numpy as jnp
from jax import lax
from jax.experimental import pallas as pl
from jax.experimental.pallas import tpu as pltpu
from jax.experimental.pallas import tpu_sc as plsc

B = 16384
NC, NS, L = 2, 16, 16
NW = NC * NS
BPW = B // NW
H = BPW // 2


@functools.partial(
    pl.kernel,
    mesh=plsc.VectorSubcoreMesh(core_axis_name="c", subcore_axis_name="s"),
    out_type=jax.ShapeDtypeStruct((B,), jnp.float32),
    scratch_types=[
        pltpu.VMEM((BPW,), jnp.int32),
        pltpu.VMEM((BPW,), jnp.float32),
        pltpu.SemaphoreType.DMA,
        pltpu.SemaphoreType.DMA,
        pltpu.SemaphoreType.DMA,
    ],
)
def _gather_sigmoid(idx_hbm, alpha_hbm, out_hbm, idx_v, vals_v,
                    sem_g0, sem_g1, sem_o):
    wid = lax.axis_index("s") * NC + lax.axis_index("c")
    base = wid * BPW

    pltpu.sync_copy(idx_hbm.at[pl.ds(base, BPW)], idx_v)

    g0 = pltpu.async_copy(alpha_hbm.at[idx_v.at[pl.ds(0, H)]],
                          vals_v.at[pl.ds(0, H)], sem_g0)
    g1 = pltpu.async_copy(alpha_hbm.at[idx_v.at[pl.ds(H, H)]],
                          vals_v.at[pl.ds(H, H)], sem_g1)

    one = jnp.full((L,), 1.0, dtype=jnp.float32)
    out_cp = []
    for h, g in enumerate((g0, g1)):
        g.wait()
        for i in range(h * H // L, (h + 1) * H // L):
            x = vals_v[pl.ds(i * L, L)]
            vals_v[pl.ds(i * L, L)] = one / (one + jnp.exp(-x))
        out_cp.append(
            pltpu.async_copy(vals_v.at[pl.ds(h * H, H)],
                             out_hbm.at[pl.ds(base + h * H, H)], sem_o))
    for c in out_cp:
        c.wait()


def kernel(idx, alpha):
    return _gather_sigmoid(idx.astype(jnp.int32), alpha)


# speedup vs baseline: 1.0228x; 1.0033x over previous
import functools

import jax
import jax.numpy as jnp
from jax import lax
from jax.experimental import pallas as pl
from jax.experimental.pallas import tpu as pltpu
from jax.experimental.pallas import tpu_sc as plsc

B = 16384
NC, NS, L = 2, 16, 16
NW = NC * NS
BPW = B // NW


@functools.partial(
    pl.kernel,
    mesh=plsc.VectorSubcoreMesh(core_axis_name="c", subcore_axis_name="s"),
    out_type=jax.ShapeDtypeStruct((B,), jnp.float32),
    scratch_types=[
        pltpu.VMEM((BPW,), jnp.int32),
        pltpu.VMEM((BPW,), jnp.float32),
        pltpu.SemaphoreType.DMA,
    ],
)
def _gather_sigmoid(idx_hbm, alpha_hbm, out_hbm, idx_v, vals_v, sem):
    wid = lax.axis_index("s") * NC + lax.axis_index("c")
    base = wid * BPW

    pltpu.sync_copy(idx_hbm.at[pl.ds(base, BPW)], idx_v)

    pltpu.async_copy(alpha_hbm.at[idx_v], vals_v, sem).wait()

    one = jnp.full((L,), 1.0, dtype=jnp.float32)
    for i in range(BPW // L):
        x = vals_v[pl.ds(i * L, L)]
        vals_v[pl.ds(i * L, L)] = one / (one + jnp.exp(-x))

    pltpu.sync_copy(vals_v, out_hbm.at[pl.ds(base, BPW)])


def kernel(idx, alpha):
    return _gather_sigmoid(idx.astype(jnp.int32), alpha)
